# Initial kernel scaffold; baseline (speedup 1.0000x reference)
import jax
import jax.numpy as jnp
from jax.experimental import pallas as pl


def kernel(batch_positives, batch_negatives, ent_emb, rel_emb, ent_proj, rel_proj):
    raise NotImplementedError("write your pallas kernel here")


# speedup vs baseline: 21.7988x; 21.7988x over previous
import functools

import jax
import jax.numpy as jnp
from jax import lax
from jax.experimental import pallas as pl
from jax.experimental.pallas import tpu as pltpu
from jax.experimental.pallas import tpu_sc as plsc

_B = 16384
_NW = 16
_CH = _B // _NW
_MARGIN = 1.0
_L = 16


def _rsqrt_nr(a):
    i = plsc.bitcast(a, jnp.int32)
    i = 0x5F3759DF - lax.shift_right_logical(i, 1)
    y = plsc.bitcast(i, jnp.float32)
    for _ in range(4):
        y = y * (1.5 - 0.5 * a * y * y)
    return y


def _renorm_scale(sq):
    inv = _rsqrt_nr(sq)
    n = sq * inv
    return jnp.where(n > 1.0, 1.0 / (n + 1e-7), 1.0)


def _sc_body(pos_hbm, neg_hbm, tbl_hbm, out_hbm,
             pos_v, neg_v, tbl_v, stab_v, part_v, shared_v, red_v, out_v,
             sem0, sem1, sem2):
    wid = lax.axis_index("s")
    base3 = wid * (_CH * 3)

    cp0 = pltpu.async_copy(pos_hbm.at[pl.ds(base3, _CH * 3)], pos_v, sem0)
    cp1 = pltpu.async_copy(neg_hbm.at[pl.ds(base3, _CH * 3)], neg_v, sem1)
    cp2 = pltpu.async_copy(tbl_hbm, tbl_v, sem2)

    cp2.wait()
    lane = lax.iota(jnp.int32, _L)
    h = lax.shift_right_logical(lane, 2) & 1
    r = lax.shift_right_logical(lane, 1) & 1
    t = lane & 1
    flat = tbl_v

    def ent_c(idx):
        e0 = plsc.load_gather(flat, [idx * _L + 0])
        e1 = plsc.load_gather(flat, [idx * _L + 1])
        p0 = plsc.load_gather(flat, [idx * _L + 2])
        p1 = plsc.load_gather(flat, [idx * _L + 3])
        s = _renorm_scale(e0 * e0 + e1 * e1)
        return (p0 * e0 + p1 * e1) * s

    x = ent_c(h) - ent_c(t)
    re0 = plsc.load_gather(flat, [r * _L + 4])
    re1 = plsc.load_gather(flat, [r * _L + 5])
    re2 = plsc.load_gather(flat, [r * _L + 6])
    rp0 = plsc.load_gather(flat, [r * _L + 7])
    rp1 = plsc.load_gather(flat, [r * _L + 8])
    rp2 = plsc.load_gather(flat, [r * _L + 9])
    sr = _renorm_scale(re0 * re0 + re1 * re1 + re2 * re2)
    d = (jnp.abs(rp0 * x + re0 * sr) + jnp.abs(rp1 * x + re1 * sr)
         + jnp.abs(rp2 * x + re2 * sr))
    stab_v[...] = d * d

    cp0.wait()
    cp1.wait()

    def step(j, acc):
        b3 = (j * _L + lane) * 3
        ph = plsc.load_gather(pos_v, [b3])
        pr = plsc.load_gather(pos_v, [b3 + 1])
        pt = plsc.load_gather(pos_v, [b3 + 2])
        nh = plsc.load_gather(neg_v, [b3])
        nr = plsc.load_gather(neg_v, [b3 + 1])
        nt = plsc.load_gather(neg_v, [b3 + 2])
        sp = plsc.load_gather(stab_v, [ph * 4 + pr * 2 + pt])
        sn = plsc.load_gather(stab_v, [nh * 4 + nr * 2 + nt])
        return acc + jnp.maximum(sp - sn + _MARGIN, 0.0)

    acc = lax.fori_loop(0, _CH // _L, step, jnp.zeros((_L,), jnp.float32))
    part_v[...] = acc
    pltpu.sync_copy(part_v, shared_v.at[wid])
    plsc.subcore_barrier()

    @pl.when(wid == 0)
    def _():
        pltpu.sync_copy(shared_v, red_v)
        tot = red_v[0]
        for s in range(1, _NW):
            tot = tot + red_v[s]
        loss = jnp.sum(tot, axis=0) * (1.0 / _B)
        out_v[...] = jnp.full((_L,), loss, jnp.float32)
        pltpu.sync_copy(out_v, out_hbm)


@functools.cache
def _build_sc_kernel():
    return pl.kernel(
        _sc_body,
        out_type=jax.ShapeDtypeStruct((_L,), jnp.float32),
        mesh=plsc.VectorSubcoreMesh(
            core_axis_name="c", subcore_axis_name="s", num_cores=1,
            num_subcores=_NW),
        scratch_types=[
            pltpu.VMEM((_CH * 3,), jnp.int32),
            pltpu.VMEM((_CH * 3,), jnp.int32),
            pltpu.VMEM((2 * _L,), jnp.float32),
            pltpu.VMEM((_L,), jnp.float32),
            pltpu.VMEM((_L,), jnp.float32),
            pltpu.VMEM_SHARED((_NW, _L), jnp.float32),
            pltpu.VMEM((_NW, _L), jnp.float32),
            pltpu.VMEM((_L,), jnp.float32),
            pltpu.SemaphoreType.DMA,
            pltpu.SemaphoreType.DMA,
            pltpu.SemaphoreType.DMA,
        ],
        compiler_params=pltpu.CompilerParams(needs_layout_passes=False),
    )


def kernel(batch_positives, batch_negatives, ent_emb, rel_emb, ent_proj, rel_proj):
    packed = jnp.concatenate(
        [ent_emb, ent_proj, rel_emb, rel_proj,
         jnp.zeros((2, _L - 10), jnp.float32)], axis=1).reshape(-1)
    out = _build_sc_kernel()(batch_positives.reshape(-1),
                             batch_negatives.reshape(-1), packed)
    return out[0]
